# Initial kernel scaffold; baseline (speedup 1.0000x reference)
import jax
import jax.numpy as jnp
from jax.experimental import pallas as pl


def kernel(atomic_embedding, pair_indices, f_ij, d_ij, G, Wi, bi, Wj, bj, Wv, bv, res_W1, res_b1, res_W2, res_b2, gate):
    raise NotImplementedError("write your pallas kernel here")


# speedup vs baseline: 1.5634x; 1.5634x over previous
import functools

import jax
import jax.numpy as jnp
from jax import lax
from jax.experimental import pallas as pl
from jax.experimental.pallas import tpu as pltpu
from jax.experimental.pallas import tpu_sc as plsc

N_NODES = 10000
D = 128

NC = 2
NS = 16
NW = NC * NS
CH = 128
N_PAD = 10240
RPT = N_PAD // NS

_LOG2 = 0.6931471805599453


def _softplus(x):
    return jnp.logaddexp(x, 0.0)


def _node_body(emb_ref, wiT_ref, bi_ref, wjT_ref, bj_ref, xi_ref, y_ref):
    x = _softplus(emb_ref[...]) - _LOG2
    xi_ref[...] = _softplus(
        jnp.dot(x, wiT_ref[...], preferred_element_type=jnp.float32) + bi_ref[...])
    y_ref[...] = _softplus(
        jnp.dot(x, wjT_ref[...], preferred_element_type=jnp.float32) + bj_ref[...])


def _node_stage(emb, wiT, bi2, wjT, bj2):
    nb = 1000
    return pl.pallas_call(
        _node_body,
        grid=(N_NODES // nb,),
        in_specs=[
            pl.BlockSpec((nb, D), lambda i: (i, 0)),
            pl.BlockSpec((D, D), lambda i: (0, 0)),
            pl.BlockSpec((1, D), lambda i: (0, 0)),
            pl.BlockSpec((D, D), lambda i: (0, 0)),
            pl.BlockSpec((1, D), lambda i: (0, 0)),
        ],
        out_specs=[pl.BlockSpec((nb, D), lambda i: (i, 0)),
                   pl.BlockSpec((nb, D), lambda i: (i, 0))],
        out_shape=[jax.ShapeDtypeStruct((N_NODES, D), jnp.float32),
                   jax.ShapeDtypeStruct((N_NODES, D), jnp.float32)],
    )(emb, wiT, bi2, wjT, bj2)


def _fp_body(f_ref, gT_ref, out_ref):
    out_ref[...] = jnp.dot(f_ref[...], gT_ref[...],
                           preferred_element_type=jnp.float32)


def _fp_stage(f_pad, gT, e_pad):
    eb = 2048
    return pl.pallas_call(
        _fp_body,
        grid=(e_pad // eb,),
        in_specs=[pl.BlockSpec((eb, 16), lambda i: (i, 0)),
                  pl.BlockSpec((16, D), lambda i: (0, 0))],
        out_specs=pl.BlockSpec((eb, D), lambda i: (i, 0)),
        out_shape=jax.ShapeDtypeStruct((e_pad, D), jnp.float32),
    )(f_pad, gT)


def _make_edge_stage(e_pad):
    epw = e_pad // NW
    nchunk = epw // CH
    mesh = plsc.VectorSubcoreMesh(core_axis_name="c", subcore_axis_name="s")

    @functools.partial(
        pl.kernel,
        out_type=jax.ShapeDtypeStruct((NC * N_PAD, D), jnp.float32),
        mesh=mesh,
        scratch_types=[
            pltpu.VMEM((CH,), jnp.int32),
            pltpu.VMEM((CH,), jnp.int32),
            pltpu.VMEM((CH, D), jnp.float32),
            pltpu.VMEM((CH, D), jnp.float32),
            pltpu.VMEM_SHARED((N_PAD, D), jnp.float32),
            pltpu.SemaphoreType.DMA,
        ],
    )
    def edge_kernel(y_hbm, idxi_hbm, idxj_hbm, fp_hbm, zeros_hbm, out_hbm,
                    idxj_v, idxi_v, rows_v, fp_v, acc_sh, sem):
        c = lax.axis_index("c")
        s = lax.axis_index("s")
        wid = s * NC + c
        row0 = s * RPT

        pltpu.sync_copy(zeros_hbm, fp_v)
        for t in range(RPT // 128):
            pltpu.sync_copy(fp_v, acc_sh.at[pl.ds(row0 + t * 128, 128)])
        plsc.subcore_barrier()

        base0 = wid * epw

        def chunk_body(k, carry):
            base = base0 + k * CH
            pltpu.sync_copy(idxj_hbm.at[pl.ds(base, CH)], idxj_v)
            pltpu.sync_copy(idxi_hbm.at[pl.ds(base, CH)], idxi_v)
            pltpu.async_copy(y_hbm.at[idxj_v], rows_v, sem).wait()
            pltpu.sync_copy(fp_hbm.at[pl.ds(base, CH)], fp_v)

            def mul_body(i, carry2):
                for db in range(D // 16):
                    sl = pl.ds(db * 16, 16)
                    rows_v[i, sl] = rows_v[i, sl] * fp_v[i, sl]
                return carry2

            lax.fori_loop(0, CH, mul_body, 0, unroll=2)
            pltpu.sync_copy(rows_v, acc_sh.at[idxi_v], add=True)
            return carry

        lax.fori_loop(0, nchunk, chunk_body, 0)
        plsc.subcore_barrier()

        out0 = c * N_PAD + row0
        for t in range(RPT // 128):
            pltpu.sync_copy(acc_sh.at[pl.ds(row0 + t * 128, 128)], rows_v)
            pltpu.sync_copy(rows_v, out_hbm.at[pl.ds(out0 + t * 128, 128)])

    return edge_kernel


def _out_body(xi_ref, part_ref, emb_ref, w1T_ref, b1_ref, w2T_ref, b2_ref,
              wvT_ref, bv_ref, gate_ref, out_ref):
    m = xi_ref[...] + part_ref[0] + part_ref[1]
    for r in range(3):
        h = _softplus(
            jnp.dot(m, w1T_ref[r], preferred_element_type=jnp.float32)
            + b1_ref[r:r + 1, :])
        h = jnp.dot(h, w2T_ref[r], preferred_element_type=jnp.float32) \
            + b2_ref[r:r + 1, :]
        m = m + h
    v = jnp.dot(_softplus(m), wvT_ref[...],
                preferred_element_type=jnp.float32) + bv_ref[...]
    out_ref[...] = jax.nn.sigmoid(gate_ref[...]) * emb_ref[...] + v


def _out_stage(xi, parts3, emb, w1T, b1, w2T, b2, wvT, bv2, gate2):
    nb = 1000
    return pl.pallas_call(
        _out_body,
        grid=(N_NODES // nb,),
        in_specs=[
            pl.BlockSpec((nb, D), lambda i: (i, 0)),
            pl.BlockSpec((NC, nb, D), lambda i: (0, i, 0)),
            pl.BlockSpec((nb, D), lambda i: (i, 0)),
            pl.BlockSpec((3, D, D), lambda i: (0, 0, 0)),
            pl.BlockSpec((3, D), lambda i: (0, 0)),
            pl.BlockSpec((3, D, D), lambda i: (0, 0, 0)),
            pl.BlockSpec((3, D), lambda i: (0, 0)),
            pl.BlockSpec((D, D), lambda i: (0, 0)),
            pl.BlockSpec((1, D), lambda i: (0, 0)),
            pl.BlockSpec((1, D), lambda i: (0, 0)),
        ],
        out_specs=pl.BlockSpec((nb, D), lambda i: (i, 0)),
        out_shape=jax.ShapeDtypeStruct((N_NODES, D), jnp.float32),
    )(xi, parts3, emb, w1T, b1, w2T, b2, wvT, bv2, gate2)


def kernel(atomic_embedding, pair_indices, f_ij, d_ij, G, Wi, bi, Wj, bj,
           Wv, bv, res_W1, res_b1, res_W2, res_b2, gate):
    e = pair_indices.shape[1]
    e_pad = ((e + NW * CH - 1) // (NW * CH)) * (NW * CH)
    pad = e_pad - e
    idx_i = jnp.pad(pair_indices[0], (0, pad))
    idx_j = jnp.pad(pair_indices[1], (0, pad))
    f_pad = jnp.pad(f_ij, ((0, pad), (0, 0)))

    xi, y = _node_stage(atomic_embedding, Wi.T, bi.reshape(1, D),
                        Wj.T, bj.reshape(1, D))
    fprime = _fp_stage(f_pad, G.T, e_pad)

    zeros = jnp.zeros((CH, D), jnp.float32)
    parts = _make_edge_stage(e_pad)(y, idx_i, idx_j, fprime, zeros)
    parts3 = parts.reshape(NC, N_PAD, D)

    return _out_stage(xi, parts3, atomic_embedding,
                      res_W1.transpose(0, 2, 1), res_b1,
                      res_W2.transpose(0, 2, 1), res_b2,
                      Wv.T, bv.reshape(1, D), gate.reshape(1, D))


# speedup vs baseline: 2.2625x; 1.4472x over previous
import functools

import jax
import jax.numpy as jnp
from jax import lax
from jax.experimental import pallas as pl
from jax.experimental.pallas import tpu as pltpu
from jax.experimental.pallas import tpu_sc as plsc

N_NODES = 10000
D = 128

NC = 2
NS = 16
NW = NC * NS
CH = 64
N_PAD = 10240
RPT = N_PAD // NS

_LOG2 = 0.6931471805599453


def _softplus(x):
    return jnp.logaddexp(x, 0.0)


def _node_body(emb_ref, wiT_ref, bi_ref, wjT_ref, bj_ref, xi_ref, y_ref):
    x = _softplus(emb_ref[...]) - _LOG2
    xi_ref[...] = _softplus(
        jnp.dot(x, wiT_ref[...], preferred_element_type=jnp.float32) + bi_ref[...])
    y_ref[...] = _softplus(
        jnp.dot(x, wjT_ref[...], preferred_element_type=jnp.float32) + bj_ref[...])


def _node_stage(emb, wiT, bi2, wjT, bj2):
    nb = 1000
    return pl.pallas_call(
        _node_body,
        grid=(N_NODES // nb,),
        in_specs=[
            pl.BlockSpec((nb, D), lambda i: (i, 0)),
            pl.BlockSpec((D, D), lambda i: (0, 0)),
            pl.BlockSpec((1, D), lambda i: (0, 0)),
            pl.BlockSpec((D, D), lambda i: (0, 0)),
            pl.BlockSpec((1, D), lambda i: (0, 0)),
        ],
        out_specs=[pl.BlockSpec((nb, D), lambda i: (i, 0)),
                   pl.BlockSpec((nb, D), lambda i: (i, 0))],
        out_shape=[jax.ShapeDtypeStruct((N_NODES, D), jnp.float32),
                   jax.ShapeDtypeStruct((N_NODES, D), jnp.float32)],
    )(emb, wiT, bi2, wjT, bj2)


def _fp_body(f_ref, gT_ref, out_ref):
    out_ref[...] = jnp.dot(f_ref[...], gT_ref[...],
                           preferred_element_type=jnp.float32)


def _fp_stage(f_pad, gT, e_pad):
    eb = 2048
    return pl.pallas_call(
        _fp_body,
        grid=(e_pad // eb,),
        in_specs=[pl.BlockSpec((eb, 16), lambda i: (i, 0)),
                  pl.BlockSpec((16, D), lambda i: (0, 0))],
        out_specs=pl.BlockSpec((eb, D), lambda i: (i, 0)),
        out_shape=jax.ShapeDtypeStruct((e_pad, D), jnp.float32),
    )(f_pad, gT)


def _make_edge_stage(e_pad):
    epw = e_pad // NW
    nchunk = epw // CH

    mesh = plsc.VectorSubcoreMesh(core_axis_name="c", subcore_axis_name="s")

    @functools.partial(
        pl.kernel,
        out_type=jax.ShapeDtypeStruct((NC * N_PAD, D), jnp.float32),
        mesh=mesh,
        scratch_types=[
            pltpu.VMEM((epw,), jnp.uint32),
            pltpu.VMEM((2, CH), jnp.int32),
            pltpu.VMEM((2, CH), jnp.int32),
            pltpu.VMEM((CH, D), jnp.float32),
            pltpu.VMEM((CH, D), jnp.float32),
            pltpu.VMEM((CH, D), jnp.float32),
            pltpu.VMEM((CH, D), jnp.float32),
            pltpu.VMEM_SHARED((N_PAD, D), jnp.float32),
            pltpu.SemaphoreType.DMA,
            pltpu.SemaphoreType.DMA,
        ],
    )
    def edge_kernel(y_hbm, idx_hbm, fp_hbm, zeros_hbm, out_hbm,
                    pidx_v, idxj_r, idxi_r, rows0_v, rows1_v, fp0_v, fp1_v,
                    acc_sh, sem0, sem1):
        c = lax.axis_index("c")
        s = lax.axis_index("s")
        wid = s * NC + c
        row0 = s * RPT
        base0 = wid * epw

        pltpu.sync_copy(zeros_hbm, rows0_v)
        for t in range(RPT // CH):
            pltpu.sync_copy(rows0_v, acc_sh.at[pl.ds(row0 + t * CH, CH)])

        pltpu.sync_copy(idx_hbm.at[pl.ds(base0, epw)], pidx_v)
        plsc.subcore_barrier()

        bufs = ((rows0_v, fp0_v, sem0), (rows1_v, fp1_v, sem1))

        def unpack_idx(kb, b):
            off = jnp.minimum(kb, nchunk - 1) * CH
            for g in range(CH // 16):
                u = pidx_v[pl.ds(off + g * 16, 16)]
                sl = pl.ds(g * 16, 16)
                idxj_r[b, sl] = (u >> 16).astype(jnp.int32)
                idxi_r[b, sl] = (u & 0xFFFF).astype(jnp.int32)

        def issue(kb, b):
            kc = jnp.minimum(kb, nchunk - 1)
            rows, fp, sem = bufs[b]
            pltpu.async_copy(y_hbm.at[idxj_r.at[b]], rows, sem)
            pltpu.async_copy(fp_hbm.at[pl.ds(base0 + kc * CH, CH)], fp, sem)

        def drain(kb, b):
            kc = jnp.minimum(kb, nchunk - 1)
            rows, fp, sem = bufs[b]
            pltpu.make_async_copy(y_hbm.at[idxj_r.at[b]], rows, sem).wait()
            pltpu.make_async_copy(
                fp_hbm.at[pl.ds(base0 + kc * CH, CH)], fp, sem).wait()

        unpack_idx(0, 0)
        issue(0, 0)
        unpack_idx(1, 1)
        issue(1, 1)

        def pair_body(i, carry):
            k = i * 2
            for b in range(2):
                kb = k + b
                rows, fp, sem = bufs[b]
                drain(kb, b)

                def mul_body(q, c2):
                    for db in range(D // 16):
                        sl = pl.ds(db * 16, 16)
                        rows[q, sl] = rows[q, sl] * fp[q, sl]
                    return c2

                lax.fori_loop(0, CH, mul_body, 0, unroll=4)
                pltpu.sync_copy(rows, acc_sh.at[idxi_r.at[b]], add=True)
                unpack_idx(kb + 2, b)
                issue(kb + 2, b)
            return carry

        lax.fori_loop(0, nchunk // 2, pair_body, 0)
        drain(nchunk, 0)
        drain(nchunk + 1, 1)
        plsc.subcore_barrier()

        out0 = c * N_PAD + row0
        for t in range(RPT // CH):
            pltpu.sync_copy(acc_sh.at[pl.ds(row0 + t * CH, CH)], rows0_v)
            pltpu.sync_copy(rows0_v, out_hbm.at[pl.ds(out0 + t * CH, CH)])

    return edge_kernel


def _out_body(xi_ref, part_ref, emb_ref, w1T_ref, b1_ref, w2T_ref, b2_ref,
              wvT_ref, bv_ref, gate_ref, out_ref):
    m = xi_ref[...] + part_ref[0] + part_ref[1]
    for r in range(3):
        h = _softplus(
            jnp.dot(m, w1T_ref[r], preferred_element_type=jnp.float32)
            + b1_ref[r:r + 1, :])
        h = jnp.dot(h, w2T_ref[r], preferred_element_type=jnp.float32) \
            + b2_ref[r:r + 1, :]
        m = m + h
    v = jnp.dot(_softplus(m), wvT_ref[...],
                preferred_element_type=jnp.float32) + bv_ref[...]
    out_ref[...] = jax.nn.sigmoid(gate_ref[...]) * emb_ref[...] + v


def _out_stage(xi, parts3, emb, w1T, b1, w2T, b2, wvT, bv2, gate2):
    nb = 1000
    return pl.pallas_call(
        _out_body,
        grid=(N_NODES // nb,),
        in_specs=[
            pl.BlockSpec((nb, D), lambda i: (i, 0)),
            pl.BlockSpec((NC, nb, D), lambda i: (0, i, 0)),
            pl.BlockSpec((nb, D), lambda i: (i, 0)),
            pl.BlockSpec((3, D, D), lambda i: (0, 0, 0)),
            pl.BlockSpec((3, D), lambda i: (0, 0)),
            pl.BlockSpec((3, D, D), lambda i: (0, 0, 0)),
            pl.BlockSpec((3, D), lambda i: (0, 0)),
            pl.BlockSpec((D, D), lambda i: (0, 0)),
            pl.BlockSpec((1, D), lambda i: (0, 0)),
            pl.BlockSpec((1, D), lambda i: (0, 0)),
        ],
        out_specs=pl.BlockSpec((nb, D), lambda i: (i, 0)),
        out_shape=jax.ShapeDtypeStruct((N_NODES, D), jnp.float32),
    )(xi, parts3, emb, w1T, b1, w2T, b2, wvT, bv2, gate2)


def kernel(atomic_embedding, pair_indices, f_ij, d_ij, G, Wi, bi, Wj, bj,
           Wv, bv, res_W1, res_b1, res_W2, res_b2, gate):
    e = pair_indices.shape[1]
    npc = -(-e // (NW * CH))
    npc += npc % 2
    e_pad = NW * CH * npc
    pad = e_pad - e
    packed_idx = jnp.pad(
        (pair_indices[1].astype(jnp.uint32) << 16)
        | pair_indices[0].astype(jnp.uint32), (0, pad))
    f_pad = jnp.pad(f_ij, ((0, pad), (0, 0)))

    xi, y = _node_stage(atomic_embedding, Wi.T, bi.reshape(1, D),
                        Wj.T, bj.reshape(1, D))
    fprime = _fp_stage(f_pad, G.T, e_pad)

    zeros = jnp.zeros((CH, D), jnp.float32)
    parts = _make_edge_stage(e_pad)(y, packed_idx, fprime, zeros)
    parts3 = parts.reshape(NC, N_PAD, D)

    return _out_stage(xi, parts3, atomic_embedding,
                      res_W1.transpose(0, 2, 1), res_b1,
                      res_W2.transpose(0, 2, 1), res_b2,
                      Wv.T, bv.reshape(1, D), gate.reshape(1, D))
